# Initial kernel scaffold; baseline (speedup 1.0000x reference)
import jax
import jax.numpy as jnp
from jax.experimental import pallas as pl


def kernel(x, edge, Wg0, Wl0, Ws0, b0, Wg1, Wl1, Ws1, b1, Wg2, Wl2, Ws2, b2):
    raise NotImplementedError("write your pallas kernel here")


# speedup vs baseline: 1.2340x; 1.2340x over previous
import functools

import jax
import jax.numpy as jnp
from jax import lax
from jax.experimental import pallas as pl
from jax.experimental.pallas import tpu as pltpu
from jax.experimental.pallas import tpu_sc as plsc

N = 10000
DEG = 32
D = 128

NW = 32
RPW = 320
NPAD = NW * RPW
C = 4
NCHUNK = RPW // C

_MESH = plsc.VectorSubcoreMesh(core_axis_name="c", subcore_axis_name="s")


@functools.partial(
    pl.kernel,
    mesh=_MESH,
    out_type=jax.ShapeDtypeStruct((NPAD, D), jnp.float32),
    scratch_types=[
        pltpu.VMEM((RPW * DEG,), jnp.int32),
        pltpu.VMEM((C * DEG, D), jnp.float32),
        pltpu.VMEM((C, D), jnp.float32),
        pltpu.SemaphoreType.DMA,
    ],
)
def _sc_gather_mean(table, idx_hbm, out_hbm, idx_v, rows_v, outb_v, sem):
    wid = lax.axis_index("s") * 2 + lax.axis_index("c")
    base = wid * RPW
    pltpu.sync_copy(idx_hbm.at[pl.ds(base * DEG, RPW * DEG)], idx_v)

    def chunk(ci, carry):
        pltpu.async_copy(
            table.at[idx_v.at[pl.ds(ci * (C * DEG), C * DEG)]], rows_v, sem
        ).wait()
        for r in range(C):
            for j in range(D // 16):
                sl = pl.ds(j * 16, 16)
                acc = rows_v[r * DEG, sl]
                for k in range(1, DEG):
                    acc = acc + rows_v[r * DEG + k, sl]
                outb_v[r, sl] = acc * (1.0 / DEG)
        pltpu.sync_copy(outb_v, out_hbm.at[pl.ds(base + ci * C, C)])
        return carry

    lax.fori_loop(0, NCHUNK, chunk, 0)


_MB = 2000


def _tc_z_body(h_ref, wg_ref, ws_ref, z_ref):
    w = wg_ref[...] + ws_ref[...]
    z_ref[...] = lax.dot_general(
        h_ref[...], w, (((1,), (1,)), ((), ())), preferred_element_type=jnp.float32
    )


def _tc_z(h, Wg, Ws):
    return pl.pallas_call(
        _tc_z_body,
        grid=(N // _MB,),
        in_specs=[
            pl.BlockSpec((_MB, D), lambda i: (i, 0)),
            pl.BlockSpec((D, D), lambda i: (0, 0)),
            pl.BlockSpec((D, D), lambda i: (0, 0)),
        ],
        out_specs=pl.BlockSpec((_MB, D), lambda i: (i, 0)),
        out_shape=jax.ShapeDtypeStruct((N, D), jnp.float32),
    )(h, Wg, Ws)


def _tc_out_body(z_ref, g_ref, wl_ref, b_ref, h_ref):
    a = (
        z_ref[...]
        + lax.dot_general(
            g_ref[...], wl_ref[...], (((1,), (1,)), ((), ())),
            preferred_element_type=jnp.float32,
        )
        + b_ref[...]
    )
    h_ref[...] = jnp.where(a > 0, a, jnp.exp(a) - 1.0)


def _tc_out(z, g, Wl, b):
    return pl.pallas_call(
        _tc_out_body,
        grid=(N // _MB,),
        in_specs=[
            pl.BlockSpec((_MB, D), lambda i: (i, 0)),
            pl.BlockSpec((_MB, D), lambda i: (i, 0)),
            pl.BlockSpec((D, D), lambda i: (0, 0)),
            pl.BlockSpec((1, D), lambda i: (0, 0)),
        ],
        out_specs=pl.BlockSpec((_MB, D), lambda i: (i, 0)),
        out_shape=jax.ShapeDtypeStruct((N, D), jnp.float32),
    )(z, g, Wl, b.reshape(1, D))


def kernel(x, edge, Wg0, Wl0, Ws0, b0, Wg1, Wl1, Ws1, b1, Wg2, Wl2, Ws2, b2):
    dst = edge[1]
    idx = jnp.concatenate(
        [dst, jnp.zeros((NPAD - N) * DEG, dtype=jnp.int32)]
    )
    h = x
    for Wg, Wl, Ws, b in ((Wg0, Wl0, Ws0, b0), (Wg1, Wl1, Ws1, b1), (Wg2, Wl2, Ws2, b2)):
        g = _sc_gather_mean(h, idx)[:N]
        z = _tc_z(h, Wg, Ws)
        h = _tc_out(z, g, Wl, b)
    return h
